# baseline (speedup 1.0000x reference)
import functools
import math

import jax
import jax.numpy as jnp
from jax import lax
from jax.experimental import pallas as pl
from jax.experimental.pallas import tpu as pltpu
from jax.experimental.pallas import tpu_sc as plsc

D_MODEL = 64
D_PAD = 128
S_LEN = 200
BL = 128
NG = 4
SCALE = math.sqrt(D_MODEL)
LANES = 16

_INFO = plsc.get_sparse_core_info()
_NC = _INFO.num_cores
_NS = _INFO.num_subcores
_NW = _NC * _NS


@functools.lru_cache(maxsize=None)
def _build_gather(n_b: int, vocab: int):
    assert n_b == _NW * BL
    assert S_LEN % NG == 0
    n_iters = S_LEN // NG

    mesh = plsc.VectorSubcoreMesh(core_axis_name="c", subcore_axis_name="s")

    @functools.partial(
        pl.kernel,
        mesh=mesh,
        out_type=jax.ShapeDtypeStruct((S_LEN, D_MODEL, n_b), jnp.float32),
        scratch_types=[pltpu.VMEM((S_LEN * BL,), jnp.int32)]
        + [pltpu.VMEM((BL, D_PAD), jnp.float32)] * NG
        + [pltpu.VMEM((D_MODEL, BL), jnp.float32)] * 2
        + [pltpu.SemaphoreType.DMA] * (NG + 2),
        compiler_params=pltpu.CompilerParams(needs_layout_passes=False),
    )
    def gather_kernel(idx_hbm, table_hbm, out_hbm, ibuf, *bufs_and_sems):
        gbufs = bufs_and_sems[:NG]
        tbufs = bufs_and_sems[NG:NG + 2]
        sg = bufs_and_sems[NG + 2:2 * NG + 2]
        sw = bufs_and_sems[2 * NG + 2:2 * NG + 4]

        wid = lax.axis_index("s") * _NC + lax.axis_index("c")

        pltpu.sync_copy(idx_hbm.at[pl.ds(wid * (S_LEN * BL), S_LEN * BL)],
                        ibuf)

        def fire_gather(s, slot):
            pltpu.async_copy(table_hbm.at[ibuf.at[pl.ds(s * BL, BL)]],
                             gbufs[slot], sg[slot])

        def wait_gather(slot):
            pltpu.make_async_copy(table_hbm.at[pl.ds(0, BL)], gbufs[slot],
                                  sg[slot]).wait()

        def transpose_scale(g, o):
            base_rows = lax.iota(jnp.int32, LANES)

            @plsc.parallel_loop(0, D_MODEL, unroll=8)
            def _(c):
                cols = jnp.full((LANES,), c, jnp.int32)
                for i in range(BL // LANES):
                    v = plsc.load_gather(g, [base_rows + i * LANES, cols])
                    o[c, pl.ds(i * LANES, LANES)] = v * SCALE

        def fire_write(s, tslot):
            pltpu.async_copy(tbufs[tslot],
                             out_hbm.at[s, :, pl.ds(wid * BL, BL)],
                             sw[tslot])

        def wait_write(tslot):
            pltpu.make_async_copy(table_hbm.at[pl.ds(0, D_MODEL)],
                                  tbufs[tslot], sw[tslot]).wait()

        for s in range(NG - 1):
            fire_gather(s, s)

        def loop_body(t, carry):
            for p in range(NG):
                c = NG * t + p
                nxt = (p + NG - 1) % NG

                @pl.when(c + NG - 1 < S_LEN)
                def _():
                    fire_gather(c + NG - 1, nxt)

                wait_gather(p)
                tslot = p % 2
                if p < 2:
                    @pl.when(t > 0)
                    def _():
                        wait_write(tslot)
                else:
                    wait_write(tslot)
                transpose_scale(gbufs[p], tbufs[tslot])
                fire_write(c, tslot)
            return carry

        lax.fori_loop(0, n_iters, loop_body, 0)

        wait_write(0)
        wait_write(1)

    return gather_kernel


def kernel(x, table):
    n_b, s = x.shape
    vocab, d = table.shape
    assert d == D_MODEL and s == S_LEN and n_b == _NW * BL
    xw = (x.astype(jnp.int32)
          .reshape(_NW, BL, S_LEN)
          .transpose(0, 2, 1)
          .reshape(-1))
    tpad = jnp.pad(table, ((0, 0), (0, D_PAD - D_MODEL)))
    outT = _build_gather(n_b, vocab)(xw, tpad)
    return outT.transpose(2, 0, 1)
